# Initial kernel scaffold; baseline (speedup 1.0000x reference)
import jax
import jax.numpy as jnp
from jax.experimental import pallas as pl


def kernel(x, A, B_ssm, W_imp, W_router, compress_neurons, knowledge_K, knowledge_V):
    raise NotImplementedError("write your pallas kernel here")


# speedup vs baseline: 19.1964x; 19.1964x over previous
import functools
import math

import jax
import jax.numpy as jnp
from jax import lax
from jax.experimental import pallas as pl
from jax.experimental.pallas import tpu as pltpu
from jax.experimental.pallas import tpu_sc as plsc

B = 2
S = 2048
D = 2048
RANK = 128
KR = 128
N_COMPRESS = 64
N_KNOW = 16384
STATE = 64
TOPK = 8

CHUNK = 128
N_CHUNKS = S // CHUNK
TT = 256
NEG = -3.4e38


def _ssm_router_body(x_ref, wcat_ref, xwrl_ref, u_ref):
    xc = x_ref[...].reshape(B * CHUNK, D)
    y = jnp.dot(xc, wcat_ref[...], preferred_element_type=jnp.float32)
    xwrl_ref[...] = y[:, STATE:STATE + 128].reshape(B, CHUNK, 128)
    u_ref[...] = y[:, 0:STATE].reshape(B, CHUNK, STATE)


def _stage_a(x, wcat):
    return pl.pallas_call(
        _ssm_router_body,
        grid=(N_CHUNKS,),
        in_specs=[
            pl.BlockSpec((B, CHUNK, D), lambda c: (0, c, 0)),
            pl.BlockSpec((D, 192), lambda c: (0, 0)),
        ],
        out_specs=[
            pl.BlockSpec((B, CHUNK, 128), lambda c: (0, c, 0)),
            pl.BlockSpec((B, CHUNK, STATE), lambda c: (0, c, 0)),
        ],
        out_shape=[
            jax.ShapeDtypeStruct((B, S, 128), jnp.float32),
            jax.ShapeDtypeStruct((B, S, STATE), jnp.float32),
        ],
    )(x, wcat)


def _recur_body(u_ref, a_ref, hfin_ref, kflat_ref, a128_ref, h_ref):
    c = pl.program_id(0)

    @pl.when(c == 0)
    def _init():
        nrows = CHUNK * STATE
        kflat_ref[nrows - STATE:nrows, :] = jnp.eye(STATE, dtype=jnp.float32)
        asq = a_ref[...]
        size = STATE
        for _ in range(7):
            bot = kflat_ref[nrows - size:nrows, :]
            kflat_ref[nrows - 2 * size:nrows - size, :] = jnp.dot(
                bot, asq, preferred_element_type=jnp.float32)
            asq = jnp.dot(asq, asq, preferred_element_type=jnp.float32)
            size *= 2
        a128_ref[...] = asq
        h_ref[...] = jnp.zeros((8, STATE), jnp.float32)

    contrib = jnp.dot(u_ref[...], kflat_ref[...],
                      preferred_element_type=jnp.float32)
    h_old = h_ref[0:B, :]
    h_new = jnp.dot(h_old, a128_ref[...],
                    preferred_element_type=jnp.float32) + contrib
    h_ref[0:B, :] = h_new

    @pl.when(c == N_CHUNKS - 1)
    def _fin():
        hfin_ref[...] = h_ref[0:B, :]


def _stage_a2(u2, a_mat):
    return pl.pallas_call(
        _recur_body,
        grid=(N_CHUNKS,),
        in_specs=[
            pl.BlockSpec((B, CHUNK * STATE), lambda c: (0, c)),
            pl.BlockSpec((STATE, STATE), lambda c: (0, 0)),
        ],
        out_specs=pl.BlockSpec((B, STATE), lambda c: (0, 0)),
        out_shape=jax.ShapeDtypeStruct((B, STATE), jnp.float32),
        scratch_shapes=[
            pltpu.VMEM((CHUNK * STATE, STATE), jnp.float32),
            pltpu.VMEM((STATE, STATE), jnp.float32),
            pltpu.VMEM((8, STATE), jnp.float32),
        ],
    )(u2, a_mat)


def _mix_body(xwrl_ref, hfin_ref, cn_ref, imp_ref, nw_ref, sc_ref, nw_s):
    j = pl.program_id(0)

    @pl.when(j == 0)
    def _head():
        xw = xwrl_ref[..., 0:N_COMPRESS]
        rl = xwrl_ref[..., N_COMPRESS:128]
        hfin = hfin_ref[...]
        implog = jnp.sum(xw * hfin[:, None, :], axis=-1)
        imp = jax.nn.softmax(implog, axis=-1)
        pref = jax.nn.softmax(rl, axis=-1)
        nw = jnp.sum(imp[:, :, None] * pref, axis=1)
        nw = nw / (jnp.sum(nw, axis=-1, keepdims=True) + 1e-8)
        imp_ref[...] = imp
        nw_ref[...] = nw
        nw_s[0:B, :] = nw

    nw = nw_s[0:B, :]
    sc_ref[...] = jnp.dot(nw, cn_ref[...], preferred_element_type=jnp.float32)


def _stage_c(xwrl, hfin, cn2):
    n_cn_chunks = 16
    cn_blk = (D * RANK) // n_cn_chunks
    return pl.pallas_call(
        _mix_body,
        grid=(n_cn_chunks,),
        in_specs=[
            pl.BlockSpec((B, S, 128), lambda j: (0, 0, 0)),
            pl.BlockSpec((B, STATE), lambda j: (0, 0)),
            pl.BlockSpec((N_COMPRESS, cn_blk), lambda j: (0, j)),
        ],
        out_specs=[
            pl.BlockSpec((B, S), lambda j: (0, 0)),
            pl.BlockSpec((B, N_COMPRESS), lambda j: (0, 0)),
            pl.BlockSpec((B, cn_blk), lambda j: (0, j)),
        ],
        out_shape=[
            jax.ShapeDtypeStruct((B, S), jnp.float32),
            jax.ShapeDtypeStruct((B, N_COMPRESS), jnp.float32),
            jax.ShapeDtypeStruct((B, D * RANK), jnp.float32),
        ],
        scratch_shapes=[pltpu.VMEM((8, N_COMPRESS), jnp.float32)],
    )(xwrl, hfin, cn2)


def _score_topk_body(x_ref, sc_ref, k_ref, idx_ref, w_ref, s_ref):
    q = jnp.dot(x_ref[0], sc_ref[0], preferred_element_type=jnp.float32)
    scores = lax.dot_general(q, k_ref[...], (((1,), (1,)), ((), ())),
                             preferred_element_type=jnp.float32)
    s_ref[...] = scores / math.sqrt(KR)

    iota = lax.broadcasted_iota(jnp.int32, (TT, N_KNOW), 1)
    vals = []
    idxs = []
    for _ in range(TOPK):
        s = s_ref[...]
        m = jnp.max(s, axis=1, keepdims=True)
        idx = jnp.min(jnp.where(s == m, iota, N_KNOW), axis=1, keepdims=True)
        vals.append(m)
        idxs.append(idx)
        s_ref[...] = jnp.where(iota == idx, NEG, s)
    topv = jnp.concatenate(vals, axis=1)
    topi = jnp.concatenate(idxs, axis=1)
    w = jax.nn.softmax(topv, axis=-1)
    idx_ref[...] = topi.reshape(1, TT, TOPK)
    w_ref[...] = w.reshape(1, TT, TOPK)


def _stage_d(x, sc3, know_k):
    n_tiles = S // TT
    return pl.pallas_call(
        _score_topk_body,
        grid=(B, n_tiles),
        in_specs=[
            pl.BlockSpec((1, TT, D), lambda b, t: (b, t, 0)),
            pl.BlockSpec((1, D, RANK), lambda b, t: (b, 0, 0)),
            pl.BlockSpec((N_KNOW, KR), lambda b, t: (0, 0)),
        ],
        out_specs=[
            pl.BlockSpec((1, TT, TOPK), lambda b, t: (b, t, 0)),
            pl.BlockSpec((1, TT, TOPK), lambda b, t: (b, t, 0)),
        ],
        out_shape=[
            jax.ShapeDtypeStruct((B, S, TOPK), jnp.int32),
            jax.ShapeDtypeStruct((B, S, TOPK), jnp.float32),
        ],
        scratch_shapes=[pltpu.VMEM((TT, N_KNOW), jnp.float32)],
    )(x, sc3, know_k)


_SC_CT = 4
_SC_NW = 32
_TOK_PER_W = (B * S) // _SC_NW


def _gather_mix_body(kv_hbm, idx_hbm, wb_hbm, out_hbm,
                     idx_v, wb_v, rows_v, out_v, sem):
    wid = lax.axis_index("s") * 2 + lax.axis_index("c")

    def chunk(t, carry):
        base_tok = wid * _TOK_PER_W + t * _SC_CT
        base_e = pl.multiple_of(base_tok * TOPK, _SC_CT * TOPK)
        pltpu.sync_copy(idx_hbm.at[pl.ds(base_e, _SC_CT * TOPK)], idx_v)
        pltpu.sync_copy(wb_hbm.at[pl.ds(base_e, _SC_CT * TOPK)], wb_v)
        pltpu.async_copy(kv_hbm.at[idx_v], rows_v, sem).wait()
        for tl in range(_SC_CT):
            ws = [wb_v[tl * TOPK + kk, :] for kk in range(TOPK)]
            def col(ci, c2):
                off = pl.ds(ci * 16, 16)
                acc = ws[0] * rows_v[tl * TOPK + 0, off]
                for kk in range(1, TOPK):
                    acc = acc + ws[kk] * rows_v[tl * TOPK + kk, off]
                out_v[tl, off] = acc
                return c2
            lax.fori_loop(0, D // 16, col, 0)
        pltpu.sync_copy(out_v, out_hbm.at[pl.ds(base_tok, _SC_CT)])
        return carry

    lax.fori_loop(0, _TOK_PER_W // _SC_CT, chunk, 0)


def _stage_e(know_v, idx_flat, w_bcast):
    mesh = plsc.VectorSubcoreMesh(core_axis_name="c", subcore_axis_name="s")
    f = pl.kernel(
        _gather_mix_body,
        out_type=jax.ShapeDtypeStruct((B * S, D), jnp.float32),
        mesh=mesh,
        scratch_types=[
            pltpu.VMEM((_SC_CT * TOPK,), jnp.int32),
            pltpu.VMEM((_SC_CT * TOPK, 16), jnp.float32),
            pltpu.VMEM((_SC_CT * TOPK, D), jnp.float32),
            pltpu.VMEM((_SC_CT, D), jnp.float32),
            pltpu.SemaphoreType.DMA,
        ],
    )
    return f(know_v, idx_flat, w_bcast)


def kernel(x, A, B_ssm, W_imp, W_router, compress_neurons,
           knowledge_K, knowledge_V):
    wcat = jnp.concatenate([B_ssm, W_imp, W_router.T], axis=1)
    xwrl, u = _stage_a(x, wcat)
    hfin = _stage_a2(u.reshape(B, S * STATE), A)
    cn2 = compress_neurons.reshape(N_COMPRESS, D * RANK)
    imp, nw, sc2 = _stage_c(xwrl, hfin, cn2)
    sc3 = sc2.reshape(B, D, RANK)
    topi, topw = _stage_d(x, sc3, knowledge_K)
    w_bcast = jnp.broadcast_to(topw.reshape(B * S * TOPK, 1),
                               (B * S * TOPK, 16))
    out_flat = _stage_e(knowledge_V,
                        topi.reshape(B * S * TOPK),
                        w_bcast)
    output = out_flat.reshape(B, S, D)
    return (output, imp, nw, topi, topw)
